# baseline (speedup 1.0000x reference)
import jax
import jax.numpy as jnp
from jax.experimental import pallas as pl
from jax.experimental.pallas import tpu as pltpu
from jax.experimental.pallas import tpu_sc as plsc

N = 2048
C = 128
OUT = 128
B = 8
R = 128
NBLK = N // R

_F32 = jnp.float32
_BF16 = jnp.bfloat16


def _tc_body(x0, x1, x2, seg, w00, w10, w01, w11, w21, w12, w22,
             lw0, lw1, lw2,
             aup0, inc1, inc1t, adn1, aup1, inc2, inc2t, adn2,
             out,
             t00, t10, t01, t11, t21, t12, t22, acc0, acc1, acc2):
    i = pl.program_id(0)

    @pl.when(i == 0)
    def _init():
        x0b = x0[...].astype(_BF16)
        x1b = x1[...].astype(_BF16)
        x2b = x2[...].astype(_BF16)
        t00[...] = jnp.dot(x0b, w00[...].astype(_BF16),
                           preferred_element_type=_F32).astype(_BF16)
        t10[...] = jnp.dot(x1b, w10[...].astype(_BF16),
                           preferred_element_type=_F32).astype(_BF16)
        t01[...] = jnp.dot(x0b, w01[...].astype(_BF16),
                           preferred_element_type=_F32).astype(_BF16)
        t11[...] = jnp.dot(x1b, w11[...].astype(_BF16),
                           preferred_element_type=_F32).astype(_BF16)
        t21[...] = jnp.dot(x2b, w21[...].astype(_BF16),
                           preferred_element_type=_F32).astype(_BF16)
        t12[...] = jnp.dot(x1b, w12[...].astype(_BF16),
                           preferred_element_type=_F32).astype(_BF16)
        t22[...] = jnp.dot(x2b, w22[...].astype(_BF16),
                           preferred_element_type=_F32).astype(_BF16)
        acc0[...] = jnp.zeros((B, C), _F32)
        acc1[...] = jnp.zeros((B, C), _F32)
        acc2[...] = jnp.zeros((B, C), _F32)

    y0 = jax.nn.sigmoid(
        jnp.dot(aup0[...].astype(_BF16), t00[...], preferred_element_type=_F32)
        + jnp.dot(inc1[...].astype(_BF16), t10[...], preferred_element_type=_F32))
    y1 = jax.nn.sigmoid(
        jnp.dot(inc1t[...].astype(_BF16), t01[...], preferred_element_type=_F32)
        + jnp.dot((adn1[...] + aup1[...]).astype(_BF16), t11[...],
                  preferred_element_type=_F32)
        + jnp.dot(inc2[...].astype(_BF16), t21[...], preferred_element_type=_F32))
    y2 = jax.nn.sigmoid(
        jnp.dot(inc2t[...].astype(_BF16), t12[...], preferred_element_type=_F32)
        + jnp.dot(adn2[...].astype(_BF16), t22[...], preferred_element_type=_F32))

    iota = jax.lax.broadcasted_iota(jnp.int32, (B, R), 0)
    oh0 = (iota == seg[0:1, pl.ds(i * R, R)]).astype(_F32)
    oh1 = (iota == seg[1:2, pl.ds(i * R, R)]).astype(_F32)
    oh2 = (iota == seg[2:3, pl.ds(i * R, R)]).astype(_F32)
    acc0[...] += jnp.dot(oh0, y0, preferred_element_type=_F32)
    acc1[...] += jnp.dot(oh1, y1, preferred_element_type=_F32)
    acc2[...] += jnp.dot(oh2, y2, preferred_element_type=_F32)

    @pl.when(i == NBLK - 1)
    def _finalize():
        out[0:B, :] = jnp.dot(acc0[...], lw0[...], preferred_element_type=_F32)
        out[B:2 * B, :] = jnp.dot(acc1[...], lw1[...],
                                  preferred_element_type=_F32)
        out[2 * B:3 * B, :] = jnp.dot(acc2[...], lw2[...],
                                      preferred_element_type=_F32)


def _full(shape):
    return pl.BlockSpec(shape, lambda i: (0,) * len(shape))


def _tc_call(x_0, x_1, x_2, seg8, W_0_0, W_1_0, W_0_1, W_1_1, W_2_1, W_1_2,
             W_2_2, lw0, lw1, lw2, aup0, inc1, inc1t, adn1, aup1, inc2,
             inc2t, adn2):
    row_spec = pl.BlockSpec((R, N), lambda i: (i, 0))
    grid_spec = pltpu.PrefetchScalarGridSpec(
        num_scalar_prefetch=0,
        grid=(NBLK,),
        in_specs=[
            _full((N, C)), _full((N, C)), _full((N, C)),
            _full((B, N)),
            _full((C, C)), _full((C, C)), _full((C, C)),
            _full((C, C)), _full((C, C)), _full((C, C)),
            _full((C, C)),
            _full((C, OUT)), _full((C, OUT)), _full((C, OUT)),
            row_spec, row_spec, row_spec, row_spec,
            row_spec, row_spec, row_spec, row_spec,
        ],
        out_specs=_full((3 * B, OUT)),
        scratch_shapes=[pltpu.VMEM((N, C), _BF16)] * 7
        + [pltpu.VMEM((B, C), _F32)] * 3,
    )
    return pl.pallas_call(
        _tc_body,
        grid_spec=grid_spec,
        out_shape=jax.ShapeDtypeStruct((3 * B, OUT), _F32),
        compiler_params=pltpu.CompilerParams(
            dimension_semantics=("arbitrary",),
        ),
    )(x_0, x_1, x_2, seg8,
      W_0_0, W_1_0, W_0_1, W_1_1, W_2_1, W_1_2, W_2_2,
      lw0, lw1, lw2,
      aup0, inc1, inc1t, adn1, aup1, inc2, inc2t, adn2)


def _sc_counts(signal_belongings):
    mesh = plsc.ScalarSubcoreMesh(axis_name="core", num_cores=2)

    @pl.kernel(
        out_type=(jax.ShapeDtypeStruct((16,), _F32),
                  jax.ShapeDtypeStruct((16,), _F32),
                  jax.ShapeDtypeStruct((16,), _F32)),
        mesh=mesh,
        scratch_types=[
            pltpu.SMEM((N,), jnp.int32),
            pltpu.SMEM((16,), _F32),
            pltpu.SemaphoreType.DMA,
        ],
    )
    def counts_kernel(seg_ref, o0_ref, o1_ref, o2_ref, row_buf, cnt, sem):
        core = jax.lax.axis_index("core")

        def do_signal(k, o_ref):
            pltpu.async_copy(seg_ref.at[pl.ds(k * N, N)], row_buf, sem).wait()

            @pl.loop(0, 16)
            def _z(j):
                cnt[j] = 0.0

            @pl.loop(0, N)
            def _c(i):
                v = row_buf[i]
                cnt[v] += 1.0

            pltpu.async_copy(cnt, o_ref, sem).wait()

        @pl.when(core == 0)
        def _core0():
            do_signal(0, o0_ref)
            do_signal(1, o1_ref)

        @pl.when(core == 1)
        def _core1():
            do_signal(2, o2_ref)

    return counts_kernel(signal_belongings.reshape(-1))


def kernel(x_0, x_1, x_2, incidence_1, incidence_2, incidence_1_transpose,
           incidence_2_transpose, adjacency_up_0_norm, adjacency_up_1_norm,
           adjacency_down_1_norm, adjacency_down_2_norm, signal_belongings,
           W_0_0, W_1_0, W_0_1, W_1_1, W_2_1, W_1_2, W_2_2,
           lw0, lb0, lw1, lb1, lw2, lb2):
    seg8 = jnp.pad(signal_belongings, ((0, B - 3), (0, 0)))

    cnts = _sc_counts(signal_belongings)
    h = _tc_call(x_0, x_1, x_2, seg8,
                 W_0_0, W_1_0, W_0_1, W_1_1, W_2_1, W_1_2, W_2_2,
                 lw0, lw1, lw2,
                 adjacency_up_0_norm, incidence_1, incidence_1_transpose,
                 adjacency_down_1_norm, adjacency_up_1_norm, incidence_2,
                 incidence_2_transpose, adjacency_down_2_norm)

    c0, c1, c2 = (jnp.maximum(x[:B], 1.0) for x in cnts)
    m0 = h[0:B, :] / c0[:, None]
    m1 = h[B:2 * B, :] / c1[:, None]
    m2 = h[2 * B:3 * B, :] / c2[:, None]
    return (m0 + m1 + m2 + (lb0 + lb1 + lb2)[None, :]) / 3.0


# speedup vs baseline: 1.3833x; 1.3833x over previous
import jax
import jax.numpy as jnp
from jax.experimental import pallas as pl
from jax.experimental.pallas import tpu as pltpu

N = 2048
C = 128
OUT = 128
B = 8
R = 128
NBLK = N // R

_F32 = jnp.float32
_BF16 = jnp.bfloat16


def _body(x0, x1, x2, seg, w00, w10, w01, w11, w21, w12, w22,
          lw0, lw1, lw2, lbs,
          aup0, inc1, inc1t, adn1, aup1, inc2, inc2t, adn2,
          out,
          t00, t10, t01, t11, t21, t12, t22, acc0, acc1, acc2):
    i = pl.program_id(0)

    @pl.when(i == 0)
    def _init():
        x0b = x0[...].astype(_BF16)
        x1b = x1[...].astype(_BF16)
        x2b = x2[...].astype(_BF16)
        t00[...] = jnp.dot(x0b, w00[...].astype(_BF16),
                           preferred_element_type=_F32).astype(_BF16)
        t10[...] = jnp.dot(x1b, w10[...].astype(_BF16),
                           preferred_element_type=_F32).astype(_BF16)
        t01[...] = jnp.dot(x0b, w01[...].astype(_BF16),
                           preferred_element_type=_F32).astype(_BF16)
        t11[...] = jnp.dot(x1b, w11[...].astype(_BF16),
                           preferred_element_type=_F32).astype(_BF16)
        t21[...] = jnp.dot(x2b, w21[...].astype(_BF16),
                           preferred_element_type=_F32).astype(_BF16)
        t12[...] = jnp.dot(x1b, w12[...].astype(_BF16),
                           preferred_element_type=_F32).astype(_BF16)
        t22[...] = jnp.dot(x2b, w22[...].astype(_BF16),
                           preferred_element_type=_F32).astype(_BF16)
        acc0[...] = jnp.zeros((B, C), _F32)
        acc1[...] = jnp.zeros((B, C), _F32)
        acc2[...] = jnp.zeros((B, C), _F32)

    y0 = jax.nn.sigmoid(
        jnp.dot(aup0[...].astype(_BF16), t00[...], preferred_element_type=_F32)
        + jnp.dot(inc1[...].astype(_BF16), t10[...], preferred_element_type=_F32))
    y1 = jax.nn.sigmoid(
        jnp.dot(inc1t[...].astype(_BF16), t01[...], preferred_element_type=_F32)
        + jnp.dot((adn1[...] + aup1[...]).astype(_BF16), t11[...],
                  preferred_element_type=_F32)
        + jnp.dot(inc2[...].astype(_BF16), t21[...], preferred_element_type=_F32))
    y2 = jax.nn.sigmoid(
        jnp.dot(inc2t[...].astype(_BF16), t12[...], preferred_element_type=_F32)
        + jnp.dot(adn2[...].astype(_BF16), t22[...], preferred_element_type=_F32))

    iota = jax.lax.broadcasted_iota(jnp.int32, (B, R), 0)
    oh0 = (iota == seg[0:1, pl.ds(i * R, R)]).astype(_F32)
    oh1 = (iota == seg[1:2, pl.ds(i * R, R)]).astype(_F32)
    oh2 = (iota == seg[2:3, pl.ds(i * R, R)]).astype(_F32)
    acc0[...] += jnp.dot(oh0, y0, preferred_element_type=_F32)
    acc1[...] += jnp.dot(oh1, y1, preferred_element_type=_F32)
    acc2[...] += jnp.dot(oh2, y2, preferred_element_type=_F32)

    @pl.when(i == NBLK - 1)
    def _finalize():
        iota_n = jax.lax.broadcasted_iota(jnp.int32, (B, N), 0)
        c0 = jnp.sum((iota_n == seg[0:1, :]).astype(_F32), axis=1, keepdims=True)
        c1 = jnp.sum((iota_n == seg[1:2, :]).astype(_F32), axis=1, keepdims=True)
        c2 = jnp.sum((iota_n == seg[2:3, :]).astype(_F32), axis=1, keepdims=True)
        m0 = jnp.dot(acc0[...] / jnp.maximum(c0, 1.0), lw0[...],
                     preferred_element_type=_F32)
        m1 = jnp.dot(acc1[...] / jnp.maximum(c1, 1.0), lw1[...],
                     preferred_element_type=_F32)
        m2 = jnp.dot(acc2[...] / jnp.maximum(c2, 1.0), lw2[...],
                     preferred_element_type=_F32)
        out[...] = (m0 + m1 + m2
                    + lbs[0:1, :] + lbs[1:2, :] + lbs[2:3, :]) / 3.0


def _full(shape):
    return pl.BlockSpec(shape, lambda i: (0,) * len(shape))


def kernel(x_0, x_1, x_2, incidence_1, incidence_2, incidence_1_transpose,
           incidence_2_transpose, adjacency_up_0_norm, adjacency_up_1_norm,
           adjacency_down_1_norm, adjacency_down_2_norm, signal_belongings,
           W_0_0, W_1_0, W_0_1, W_1_1, W_2_1, W_1_2, W_2_2,
           lw0, lb0, lw1, lb1, lw2, lb2):
    seg8 = jnp.pad(signal_belongings, ((0, B - 3), (0, 0)))
    lbs = jnp.pad(jnp.stack([lb0, lb1, lb2]), ((0, B - 3), (0, 0)))

    row_spec = pl.BlockSpec((R, N), lambda i: (i, 0))
    grid_spec = pltpu.PrefetchScalarGridSpec(
        num_scalar_prefetch=0,
        grid=(NBLK,),
        in_specs=[
            _full((N, C)), _full((N, C)), _full((N, C)),
            _full((B, N)),
            _full((C, C)), _full((C, C)), _full((C, C)),
            _full((C, C)), _full((C, C)), _full((C, C)),
            _full((C, C)),
            _full((C, OUT)), _full((C, OUT)), _full((C, OUT)),
            _full((B, OUT)),
            row_spec, row_spec, row_spec, row_spec,
            row_spec, row_spec, row_spec, row_spec,
        ],
        out_specs=_full((B, OUT)),
        scratch_shapes=[pltpu.VMEM((N, C), _BF16)] * 7
        + [pltpu.VMEM((B, C), _F32)] * 3,
    )
    return pl.pallas_call(
        _body,
        grid_spec=grid_spec,
        out_shape=jax.ShapeDtypeStruct((B, OUT), _F32),
        compiler_params=pltpu.CompilerParams(
            dimension_semantics=("arbitrary",),
        ),
    )(x_0, x_1, x_2, seg8,
      W_0_0, W_1_0, W_0_1, W_1_1, W_2_1, W_1_2, W_2_2,
      lw0, lw1, lw2, lbs,
      adjacency_up_0_norm, incidence_1, incidence_1_transpose,
      adjacency_down_1_norm, adjacency_up_1_norm, incidence_2,
      incidence_2_transpose, adjacency_down_2_norm)
